# baseline (speedup 1.0000x reference)
import functools

import jax
import jax.numpy as jnp
from jax.experimental import pallas as pl
from jax.experimental.pallas import tpu as pltpu

TOPK = 8
NUM_EXPERTS = 64
NEG = -jnp.inf


def _router_kernel(x1_ref, x2_ref, w1_ref, w2_ref, weights_ref, indices_ref):
    logits_t = jax.lax.dot_general(
        w1_ref[...], x1_ref[...],
        (((1,), (1,)), ((), ())),
        preferred_element_type=jnp.float32,
    ) + jax.lax.dot_general(
        w2_ref[...], x2_ref[...],
        (((1,), (1,)), ((), ())),
        preferred_element_type=jnp.float32,
    )
    br = logits_t.shape[1]
    inv_row = jax.lax.broadcasted_iota(jnp.int32, (NUM_EXPERTS, br), 0)
    inv_row = (NUM_EXPERTS - 1) - inv_row
    inv_row_f = inv_row.astype(jnp.float32)

    work = logits_t
    vals = []
    idxs = []
    for _ in range(TOPK):
        m = jnp.max(work, axis=0, keepdims=True)
        t = jnp.where(work == m, inv_row_f, -1.0)
        r = jnp.max(t, axis=0, keepdims=True)
        vals.append(m)
        idxs.append(r)
        work = jnp.where(t == r, NEG, work)

    v = jnp.concatenate(vals, axis=0)
    i = (NUM_EXPERTS - 1) - jnp.concatenate(idxs, axis=0).astype(jnp.int32)
    e = jnp.exp(v - v[0:1, :])
    w = e / jnp.sum(e, axis=0, keepdims=True)
    weights_ref[...] = w.T
    indices_ref[...] = i.T


@functools.partial(jax.jit, static_argnames=())
def kernel(x, W):
    T, H = x.shape
    BR = 1024
    grid = (T // BR,)
    weights, indices = pl.pallas_call(
        _router_kernel,
        grid=grid,
        in_specs=[
            pl.BlockSpec((BR, H // 2), lambda r: (r, 0)),
            pl.BlockSpec((BR, H // 2), lambda r: (r, 1)),
            pl.BlockSpec((NUM_EXPERTS, H // 2), lambda r: (0, 0)),
            pl.BlockSpec((NUM_EXPERTS, H // 2), lambda r: (0, 1)),
        ],
        out_specs=[
            pl.BlockSpec((BR, TOPK), lambda r: (r, 0)),
            pl.BlockSpec((BR, TOPK), lambda r: (r, 0)),
        ],
        out_shape=[
            jax.ShapeDtypeStruct((T, TOPK), jnp.float32),
            jax.ShapeDtypeStruct((T, TOPK), jnp.int32),
        ],
        compiler_params=pltpu.CompilerParams(
            vmem_limit_bytes=100 * 1024 * 1024,
        ),
    )(x, x, W, W)
    return (weights, indices)
